# Initial kernel scaffold; baseline (speedup 1.0000x reference)
import jax
import jax.numpy as jnp
from jax.experimental import pallas as pl


def kernel(x, edge_index, edge_attr, W1, b1, W2, b2, g1_Wrel, g1_brel, g1_Wroot, g2_Wrel, g2_brel, g2_Wroot, mu_W, mu_b, lv_W, lv_b, dW1, db1, dW2, db2):
    raise NotImplementedError("write your pallas kernel here")


# speedup vs baseline: 3.9299x; 3.9299x over previous
import functools

import jax
import jax.numpy as jnp
from jax import lax
from jax.experimental import pallas as pl
from jax.experimental.pallas import tpu as pltpu
from jax.experimental.pallas import tpu_sc as plsc

N = 10000
E = 320000
D_IN = 128
H = 128
LAT = 32

NC = 2
NS = 16
NW = NC * NS
EPW = E // NW
CHUNK = 80
NCHUNK = EPW // CHUNK
ROWS_PT = N // NS

_PREC = lax.Precision.HIGHEST


def _matmul_t(a, w):
    return lax.dot_general(a, w, (((1,), (1,)), ((), ())), precision=_PREC)


def _sc_agg_body(h_hbm, ei_hbm, ew_hbm, out_hbm,
                 agg_sh, src_v, dst_v, ew_v, rows_v, gsem):
    cid = lax.axis_index("c")
    sid = lax.axis_index("s")
    wid = sid * NC + cid

    zero16 = jnp.zeros((16,), jnp.float32)

    def _zero_row(r, _):
        for j in range(H // 16):
            rows_v[r, pl.ds(16 * j, 16)] = zero16
        return _

    lax.fori_loop(0, CHUNK, _zero_row, 0, unroll=False)

    def _zero_copy(k, carry):
        c = sid + NS * k

        @pl.when(c < N // CHUNK)
        def _do():
            pltpu.sync_copy(rows_v, agg_sh.at[pl.ds(c * CHUNK, CHUNK)])

        return carry

    lax.fori_loop(0, pl.cdiv(N // CHUNK, NS), _zero_copy, 0, unroll=False)

    plsc.subcore_barrier()

    def _chunk(i, _):
        base = wid * EPW + i * CHUNK
        pltpu.sync_copy(ei_hbm.at[pl.ds(base, CHUNK)], src_v)
        pltpu.sync_copy(ei_hbm.at[pl.ds(E + base, CHUNK)], dst_v)
        pltpu.sync_copy(ew_hbm.at[pl.ds(base, CHUNK)], ew_v)
        pltpu.async_copy(h_hbm.at[src_v], rows_v, gsem).wait()

        def _scale(g, _c):
            ew16 = ew_v[pl.ds(16 * g, 16)]
            for l in range(16):
                e = 16 * g + l
                w = ew16[l]
                for j in range(H // 16):
                    rows_v[e, pl.ds(16 * j, 16)] = rows_v[e, pl.ds(16 * j, 16)] * w
            return _c

        lax.fori_loop(0, CHUNK // 16, _scale, 0, unroll=False)
        pltpu.sync_copy(rows_v, agg_sh.at[dst_v], add=True)
        return _

    lax.fori_loop(0, NCHUNK, _chunk, 0, unroll=False)

    plsc.subcore_barrier()

    def _out_copy(k, carry):
        c = sid + NS * k

        @pl.when(c < N // CHUNK)
        def _do():
            pltpu.sync_copy(agg_sh.at[pl.ds(c * CHUNK, CHUNK)],
                            out_hbm.at[cid, pl.ds(c * CHUNK, CHUNK)])

        return carry

    lax.fori_loop(0, pl.cdiv(N // CHUNK, NS), _out_copy, 0, unroll=False)


_sc_agg = functools.partial(
    pl.kernel,
    out_type=jax.ShapeDtypeStruct((NC, N, H), jnp.float32),
    mesh=plsc.VectorSubcoreMesh(core_axis_name="c", subcore_axis_name="s"),
    scratch_types=[
        pltpu.VMEM_SHARED((N, H), jnp.float32),
        pltpu.VMEM((CHUNK,), jnp.int32),
        pltpu.VMEM((CHUNK,), jnp.int32),
        pltpu.VMEM((CHUNK,), jnp.float32),
        pltpu.VMEM((CHUNK, H), jnp.float32),
        pltpu.SemaphoreType.DMA,
    ],
)(_sc_agg_body)


def _enc_body(x_ref, w1_ref, b1_ref, w2_ref, b2_ref, o_ref):
    h = jnp.maximum(_matmul_t(x_ref[...], w1_ref[...]) + b1_ref[...], 0.0)
    o_ref[...] = jnp.maximum(_matmul_t(h, w2_ref[...]) + b2_ref[...], 0.0)


def _conv_dense_body(aggp_ref, h_ref, wrel_ref, brel_ref, wroot_ref, o_ref):
    agg = aggp_ref[0] + aggp_ref[1]
    o_ref[...] = jnp.maximum(
        _matmul_t(agg, wrel_ref[...]) + brel_ref[...]
        + _matmul_t(h_ref[...], wroot_ref[...]), 0.0)


def _tail_body(aggp_ref, h_ref, wrel_ref, brel_ref, wroot_ref,
               muw_ref, mub_ref, lvw_ref, lvb_ref, eps_ref,
               dw1_ref, db1_ref, dw2_ref, db2_ref,
               xr_ref, mu_ref, lv_ref, z_ref):
    agg = aggp_ref[0] + aggp_ref[1]
    h3 = jnp.maximum(
        _matmul_t(agg, wrel_ref[...]) + brel_ref[...]
        + _matmul_t(h_ref[...], wroot_ref[...]), 0.0)
    mu = _matmul_t(h3, muw_ref[...]) + mub_ref[...]
    lv = _matmul_t(h3, lvw_ref[...]) + lvb_ref[...]
    z = mu + eps_ref[...] * jnp.exp(0.5 * lv)
    d = jnp.maximum(_matmul_t(z, dw1_ref[...]) + db1_ref[...], 0.0)
    xr_ref[...] = _matmul_t(d, dw2_ref[...]) + db2_ref[...]
    mu_ref[...] = mu
    lv_ref[...] = lv
    z_ref[...] = z


def _tc_call(body, n_out_shapes):
    return pl.pallas_call(body, out_shape=n_out_shapes)


def kernel(x, edge_index, edge_attr, W1, b1, W2, b2,
           g1_Wrel, g1_brel, g1_Wroot, g2_Wrel, g2_brel, g2_Wroot,
           mu_W, mu_b, lv_W, lv_b, dW1, db1, dW2, db2):
    b1r = b1.reshape(1, H)
    b2r = b2.reshape(1, H)
    g1br = g1_brel.reshape(1, H)
    g2br = g2_brel.reshape(1, H)
    mubr = mu_b.reshape(1, LAT)
    lvbr = lv_b.reshape(1, LAT)
    db1r = db1.reshape(1, H)
    db2r = db2.reshape(1, D_IN)
    eps = jax.random.normal(jax.random.key(42), (N, LAT), dtype=jnp.float32)

    h1 = _tc_call(_enc_body, jax.ShapeDtypeStruct((N, H), jnp.float32))(
        x, W1, b1r, W2, b2r)

    ei_flat = edge_index.reshape(2 * E)
    agg1p = _sc_agg(h1, ei_flat, edge_attr)
    h2 = _tc_call(_conv_dense_body, jax.ShapeDtypeStruct((N, H), jnp.float32))(
        agg1p, h1, g1_Wrel, g1br, g1_Wroot)

    agg2p = _sc_agg(h2, ei_flat, edge_attr)
    BT = 2000
    full = lambda s: pl.BlockSpec(s, lambda i: (0,) * len(s))
    xr, mu, lv, z = pl.pallas_call(
        _tail_body,
        grid=(N // BT,),
        in_specs=[
            pl.BlockSpec((NC, BT, H), lambda i: (0, i, 0)),
            pl.BlockSpec((BT, H), lambda i: (i, 0)),
            full((H, H)), full((1, H)), full((H, H)),
            full((LAT, H)), full((1, LAT)), full((LAT, H)), full((1, LAT)),
            pl.BlockSpec((BT, LAT), lambda i: (i, 0)),
            full((H, LAT)), full((1, H)), full((D_IN, H)), full((1, D_IN)),
        ],
        out_specs=[
            pl.BlockSpec((BT, D_IN), lambda i: (i, 0)),
            pl.BlockSpec((BT, LAT), lambda i: (i, 0)),
            pl.BlockSpec((BT, LAT), lambda i: (i, 0)),
            pl.BlockSpec((BT, LAT), lambda i: (i, 0)),
        ],
        out_shape=(
            jax.ShapeDtypeStruct((N, D_IN), jnp.float32),
            jax.ShapeDtypeStruct((N, LAT), jnp.float32),
            jax.ShapeDtypeStruct((N, LAT), jnp.float32),
            jax.ShapeDtypeStruct((N, LAT), jnp.float32),
        ),
    )(agg2p, h2, g2_Wrel, g2br, g2_Wroot,
      mu_W, mubr, lv_W, lvbr, eps, dW1, db1r, dW2, db2r)

    return (xr, mu, lv, z)
